# Initial kernel scaffold; baseline (speedup 1.0000x reference)
import jax
import jax.numpy as jnp
from jax.experimental import pallas as pl


def kernel(input_features, centroids):
    raise NotImplementedError("write your pallas kernel here")


# speedup vs baseline: 1.9197x; 1.9197x over previous
import jax
import jax.numpy as jnp
from jax.experimental import pallas as pl

_N = 1024
_D = 64
_K = 1000
_KC = _K // 8


def _minmax_idx_kernel(f2_ref, c2_ref, min_ref, max_ref):
    inf = jnp.float32(jnp.inf)

    def col_body(ck, carry):
        lo_v, lo_i, hi_v, hi_i = carry
        rows = c2_ref[pl.ds(ck * 8, 8), :]
        kid = ck * 8 + jax.lax.broadcasted_iota(jnp.int32, (8, 128), 0)
        lt = rows < lo_v
        gt = rows > hi_v
        lo_v = jnp.where(lt, rows, lo_v)
        lo_i = jnp.where(lt, kid, lo_i)
        hi_v = jnp.where(gt, rows, hi_v)
        hi_i = jnp.where(gt, kid, hi_i)
        return lo_v, lo_i, hi_v, hi_i

    zi = jnp.zeros((8, 128), jnp.int32)
    lo_v, lo_i, hi_v, hi_i = jax.lax.fori_loop(
        0, _KC, col_body,
        (jnp.full((8, 128), inf, jnp.float32), zi,
         jnp.full((8, 128), -inf, jnp.float32), zi))

    def red(v, i, is_min):
        bv, bi = v[0:1, :], i[0:1, :]
        for r in range(1, 8):
            vv, ii = v[r:r + 1, :], i[r:r + 1, :]
            if is_min:
                better = (vv < bv) | ((vv == bv) & (ii < bi))
            else:
                better = (vv > bv) | ((vv == bv) & (ii < bi))
            bv = jnp.where(better, vv, bv)
            bi = jnp.where(better, ii, bi)
        return jnp.broadcast_to(bv, (8, 128)), jnp.broadcast_to(bi, (8, 128))

    lo_vb, lo_ib = red(lo_v, lo_i, True)
    hi_vb, hi_ib = red(hi_v, hi_i, False)

    def tile_body(t, _):
        ft = f2_ref[pl.ds(t * 8, 8), :]

        def k_body(ck, carry):
            minv, mini = carry
            crows = c2_ref[pl.ds(ck * 8, 8), :]
            for dk in range(8):
                row = jnp.broadcast_to(crows[dk:dk + 1, :], (8, 128))
                d = ft - row
                sq = d * d
                upd = sq < minv
                minv = jnp.where(upd, sq, minv)
                mini = jnp.where(upd, ck * 8 + dk, mini)
            return minv, mini

        minv, mini = jax.lax.fori_loop(
            0, _KC, k_body,
            (jnp.full((8, 128), inf, jnp.float32), jnp.zeros((8, 128), jnp.int32)))

        dl = ft - lo_vb
        dh = ft - hi_vb
        sql = dl * dl
        sqh = dh * dh
        maxi = jnp.where(sql > sqh, lo_ib,
                         jnp.where(sqh > sql, hi_ib, jnp.minimum(lo_ib, hi_ib)))
        min_ref[pl.ds(t * 8, 8), :] = mini
        max_ref[pl.ds(t * 8, 8), :] = maxi
        return 0

    jax.lax.fori_loop(0, _N // 2 // 8, tile_body, 0)


def _loss_kernel(f_ref, cpad_ref, mini_ref, maxi_ref, out_ref):
    f = f_ref[...]
    mi = mini_ref[...]
    ma = maxi_ref[...]

    def mode(idx):
        counts = jnp.zeros((_N, _D), jnp.int32)
        for r in range(_D):
            rolled = jnp.roll(idx, r, axis=1) if r else idx
            counts = counts + (rolled == idx).astype(jnp.int32)
        key = counts * 1024 + (1023 - idx)
        mkey = jnp.max(key, axis=1, keepdims=True)
        return 1023 - jnp.bitwise_and(mkey, 1023)

    mode_min = mode(mi)
    mode_max = mode(ma)

    iota_m = jax.lax.broadcasted_iota(jnp.int32, (_N, _N), 1)
    oh_p = (mode_min == iota_m).astype(jnp.float32)
    oh_n = (mode_max == iota_m).astype(jnp.float32)
    cpad = cpad_ref[...]
    pos = jnp.dot(oh_p, cpad, preferred_element_type=jnp.float32)
    neg = jnp.dot(oh_n, cpad, preferred_element_type=jnp.float32)

    eps = jnp.float32(1e-6)

    def pdist(a, b):
        d = a - b + eps
        return jnp.sqrt(jnp.sum(d * d, axis=1, keepdims=True))

    d_ap = pdist(f, pos)
    d_an = pdist(f, neg)
    d_pn = pdist(pos, neg)
    d_neg = jnp.minimum(d_an, d_pn)
    li = jnp.maximum(d_ap - d_neg + 1.0, 0.0)
    loss = jnp.sum(li) * jnp.float32(1.0 / _N)
    out_ref[...] = jnp.full((8, 128), loss, jnp.float32)


def kernel(input_features, centroids):
    f = input_features.astype(jnp.float32)
    c = centroids.astype(jnp.float32)
    f2 = f.reshape(_N // 2, 2 * _D)
    c2 = jnp.tile(c, (1, 2))

    mini2, maxi2 = pl.pallas_call(
        _minmax_idx_kernel,
        out_shape=[jax.ShapeDtypeStruct((_N // 2, 2 * _D), jnp.int32)] * 2,
    )(f2, c2)
    mini = mini2.reshape(_N, _D)
    maxi = maxi2.reshape(_N, _D)

    cpad = jnp.concatenate([c, jnp.zeros((_N - _K, _D), jnp.float32)], axis=0)
    out = pl.pallas_call(
        _loss_kernel,
        out_shape=jax.ShapeDtypeStruct((8, 128), jnp.float32),
    )(f, cpad, mini, maxi)
    return out[0, 0]


# speedup vs baseline: 4.3598x; 2.2711x over previous
import jax
import jax.numpy as jnp
from jax.experimental import pallas as pl

_N = 1024
_D = 64
_K = 1000
_KC = _K // 8
_G = 4


def _minmax_idx_kernel(f2_ref, c2_ref, min_ref, max_ref):
    inf = jnp.float32(jnp.inf)

    def col_body(ck, carry):
        lo_v, lo_i, hi_v, hi_i = carry
        rows = c2_ref[pl.ds(ck * 8, 8), :]
        kid = ck * 8 + jax.lax.broadcasted_iota(jnp.int32, (8, 128), 0)
        lt = rows < lo_v
        gt = rows > hi_v
        lo_v = jnp.where(lt, rows, lo_v)
        lo_i = jnp.where(lt, kid, lo_i)
        hi_v = jnp.where(gt, rows, hi_v)
        hi_i = jnp.where(gt, kid, hi_i)
        return lo_v, lo_i, hi_v, hi_i

    zi = jnp.zeros((8, 128), jnp.int32)
    lo_v, lo_i, hi_v, hi_i = jax.lax.fori_loop(
        0, _KC, col_body,
        (jnp.full((8, 128), inf, jnp.float32), zi,
         jnp.full((8, 128), -inf, jnp.float32), zi))

    def red(v, i, is_min):
        bv, bi = v[0:1, :], i[0:1, :]
        for r in range(1, 8):
            vv, ii = v[r:r + 1, :], i[r:r + 1, :]
            if is_min:
                better = (vv < bv) | ((vv == bv) & (ii < bi))
            else:
                better = (vv > bv) | ((vv == bv) & (ii < bi))
            bv = jnp.where(better, vv, bv)
            bi = jnp.where(better, ii, bi)
        return jnp.broadcast_to(bv, (8, 128)), jnp.broadcast_to(bi, (8, 128))

    lo_vb, lo_ib = red(lo_v, lo_i, True)
    hi_vb, hi_ib = red(hi_v, hi_i, False)

    def group_body(g, _):
        base = g * (8 * _G)
        fts = [f2_ref[pl.ds(base + t * 8, 8), :] for t in range(_G)]

        def k_body(ck, carry):
            minvs, minis = list(carry[0]), list(carry[1])
            crows = c2_ref[pl.ds(ck * 8, 8), :]
            for dk in range(8):
                row = jnp.broadcast_to(crows[dk:dk + 1, :], (8, 128))
                kidx = ck * 8 + dk
                for t in range(_G):
                    d = fts[t] - row
                    sq = d * d
                    upd = sq < minvs[t]
                    minvs[t] = jnp.where(upd, sq, minvs[t])
                    minis[t] = jnp.where(upd, kidx, minis[t])
            return tuple(minvs), tuple(minis)

        init = (tuple(jnp.full((8, 128), inf, jnp.float32) for _ in range(_G)),
                tuple(jnp.zeros((8, 128), jnp.int32) for _ in range(_G)))
        _, minis = jax.lax.fori_loop(0, _KC, k_body, init)

        for t in range(_G):
            ft = fts[t]
            dl = ft - lo_vb
            dh = ft - hi_vb
            sql = dl * dl
            sqh = dh * dh
            maxi = jnp.where(sql > sqh, lo_ib,
                             jnp.where(sqh > sql, hi_ib,
                                       jnp.minimum(lo_ib, hi_ib)))
            min_ref[pl.ds(base + t * 8, 8), :] = minis[t]
            max_ref[pl.ds(base + t * 8, 8), :] = maxi
        return 0

    jax.lax.fori_loop(0, _N // 2 // 8 // _G, group_body, 0)


def _loss_kernel(f_ref, cpad_ref, miT_ref, maT_ref, out_ref):
    f = f_ref[...]

    def mode(idxT):
        counts = jnp.zeros((_D, _N), jnp.int32)
        for r in range(_D):
            rolled = jnp.roll(idxT, r, axis=0) if r else idxT
            counts = counts + (rolled == idxT).astype(jnp.int32)
        key = counts * 1024 + (1023 - idxT)
        mkey = jnp.max(key, axis=0, keepdims=True)
        mode_row = 1023 - jnp.bitwise_and(mkey, 1023)
        return mode_row.reshape(_N, 1)

    mode_min = mode(miT_ref[...])
    mode_max = mode(maT_ref[...])

    iota_m = jax.lax.broadcasted_iota(jnp.int32, (_N, _N), 1)
    oh_p = (mode_min == iota_m).astype(jnp.float32)
    oh_n = (mode_max == iota_m).astype(jnp.float32)
    cpad = cpad_ref[...]
    pos = jnp.dot(oh_p, cpad, preferred_element_type=jnp.float32)
    neg = jnp.dot(oh_n, cpad, preferred_element_type=jnp.float32)

    eps = jnp.float32(1e-6)

    def pdist(a, b):
        d = a - b + eps
        return jnp.sqrt(jnp.sum(d * d, axis=1, keepdims=True))

    d_ap = pdist(f, pos)
    d_an = pdist(f, neg)
    d_pn = pdist(pos, neg)
    d_neg = jnp.minimum(d_an, d_pn)
    li = jnp.maximum(d_ap - d_neg + 1.0, 0.0)
    loss = jnp.sum(li) * jnp.float32(1.0 / _N)
    out_ref[...] = jnp.full((8, 128), loss, jnp.float32)


def kernel(input_features, centroids):
    f = input_features.astype(jnp.float32)
    c = centroids.astype(jnp.float32)
    f2 = f.reshape(_N // 2, 2 * _D)
    c2 = jnp.tile(c, (1, 2))

    mini2, maxi2 = pl.pallas_call(
        _minmax_idx_kernel,
        out_shape=[jax.ShapeDtypeStruct((_N // 2, 2 * _D), jnp.int32)] * 2,
    )(f2, c2)
    miT = mini2.reshape(_N, _D).T
    maT = maxi2.reshape(_N, _D).T

    cpad = jnp.concatenate([c, jnp.zeros((_N - _K, _D), jnp.float32)], axis=0)
    out = pl.pallas_call(
        _loss_kernel,
        out_shape=jax.ShapeDtypeStruct((8, 128), jnp.float32),
    )(f, cpad, miT, maT)
    return out[0, 0]
